# Initial kernel scaffold; baseline (speedup 1.0000x reference)
import jax
import jax.numpy as jnp
from jax.experimental import pallas as pl


def kernel(x, U, R, S, W_in, b_in, W_tok, b_tok, W_q, b_q, W_k, b_k, W_v, b_v, W_ih, b_ih, W_hh, b_hh, W_g1, b_g1, W_g2, b_g2, W_e1, b_e1, W_e2, b_e2, W_out, b_out):
    raise NotImplementedError("write your pallas kernel here")


# speedup vs baseline: 1.7225x; 1.7225x over previous
import math

import jax
import jax.numpy as jnp
import numpy as np
from jax.experimental import pallas as pl

B, M, T = 2, 4, 512
D_IN = 512
D_MODEL = 512
E, TOPK = 8, 2
H_EXP = 1024
TP = 128
GRU_H = 128
AK = 64
AV = 64
N_CLS = 10
N = B * M * T
SEQ = B * M
NT = M * T
NBB = NT // T
NEG = -1e30


def _pos_encoding_np(t, d):
    position = np.arange(t)[:, None].astype(np.float32)
    div = np.exp(np.arange(0, d, 2).astype(np.float32) * (-math.log(10000.0) / d))
    pe = np.zeros((t, d), dtype=np.float32)
    pe[:, 0::2] = np.sin(position * div)
    pe[:, 1::2] = np.cos(position * div)
    return pe


def _dot(a, b):
    return jnp.dot(a, b, preferred_element_type=jnp.float32)


def _pre_kernel(x_ref, pe_ref, u_ref, r_ref, s_ref,
                w_in_ref, b_in_ref, w_tok_ref, b_tok_ref, w_q_ref, b_q_ref,
                wk_t_ref, w_v_ref, b_v_ref, wih_t_ref, b_ih_ref,
                h_ref, proc_ref, gi_ref):
    i = pl.program_id(0)
    m = i % M
    h = _dot(x_ref[...], w_in_ref[...]) + b_in_ref[...] + pe_ref[...]
    h_ref[...] = h
    proc = jnp.maximum(_dot(h, w_tok_ref[...]) + b_tok_ref[...], 0.0)
    proc_ref[...] = proc
    q = _dot(proc, w_q_ref[...]) + b_q_ref[...]
    qk = _dot(q, wk_t_ref[...]) * (1.0 / math.sqrt(AK))
    a_r = qk[:, 0:1]
    a_s = qk[:, 1:2]
    rb = r_ref[0]
    sb = s_ref[0]
    scores = rb * a_r + sb * a_s
    col = jax.lax.broadcasted_iota(jnp.int32, (T, M), 1)
    scores = jnp.where(col == m, NEG, scores)
    w = jnp.exp(scores - jnp.max(scores, axis=1, keepdims=True))
    w = w / jnp.sum(w, axis=1, keepdims=True)
    wr = jnp.sum(w * rb, axis=1, keepdims=True)
    ws = jnp.sum(w * sb, axis=1, keepdims=True)
    c_u = wih_t_ref[0:1, :]
    c_rs = _dot(w_v_ref[...], wih_t_ref[1:, :])
    c_0 = _dot(b_v_ref[...], wih_t_ref[1:, :]) + b_ih_ref[...]
    gi = (u_ref[0] * c_u + wr * c_rs[0:1, :] + ws * c_rs[1:2, :] + c_0)
    gi_ref[0] = gi


def _gru_kernel(gi_ref, whh_t_ref, b_hh_ref, out_ref):
    def step(t, h):
        gi = gi_ref[t]
        gh = _dot(h, whh_t_ref[...]) + b_hh_ref[...]
        r = jax.nn.sigmoid(gi[:, :GRU_H] + gh[:, :GRU_H])
        z = jax.nn.sigmoid(gi[:, GRU_H:2 * GRU_H] + gh[:, GRU_H:2 * GRU_H])
        n = jnp.tanh(gi[:, 2 * GRU_H:] + r * gh[:, 2 * GRU_H:])
        h_new = (1.0 - z) * n + z * h
        out_ref[t] = h_new
        return h_new
    jax.lax.fori_loop(0, T, step, jnp.zeros((SEQ, GRU_H), jnp.float32))


def _route_kernel(proc_ref, gru_ref, wg1a_ref, wg1b_ref, b_g1_ref,
                  w_g2_ref, b_g2_ref, gates_ref):
    hid = jnp.maximum(_dot(proc_ref[...], wg1a_ref[...]) +
                      _dot(gru_ref[...], wg1b_ref[...]) + b_g1_ref[...], 0.0)
    logits = _dot(hid, w_g2_ref[...]) + b_g2_ref[...]
    idx = jax.lax.broadcasted_iota(jnp.int32, (T, E), 1)
    v1 = jnp.max(logits, axis=1, keepdims=True)
    i1 = jnp.min(jnp.where(logits == v1, idx, E), axis=1, keepdims=True)
    masked = jnp.where(idx == i1, NEG, logits)
    v2 = jnp.max(masked, axis=1, keepdims=True)
    i2 = jnp.min(jnp.where(masked == v2, idx, E), axis=1, keepdims=True)
    e2 = jnp.exp(v2 - v1)
    g1 = 1.0 / (1.0 + e2)
    gates_ref[...] = jnp.where(idx == i1, g1, 0.0) + jnp.where(idx == i2, e2 * g1, 0.0)


def _expert_kernel(h_ref, w1_ref, b1_ref, g_ref, s_ref):
    k = pl.program_id(2)
    eh = jnp.maximum(_dot(h_ref[...], w1_ref[0]) + b1_ref[0], 0.0)
    contrib = _dot(g_ref[0], eh)

    @pl.when(k == 0)
    def _():
        s_ref[0] = contrib

    @pl.when(k != 0)
    def _():
        s_ref[0] += contrib


def _final_kernel(h_ref, s_ref, g_ref, w2_ref, b2_ref, wout_ref, bout_ref, out_ref):
    hv = h_ref[...]
    hm0 = jnp.sum(hv[:NT], axis=0, keepdims=True)
    hm1 = jnp.sum(hv[NT:], axis=0, keepdims=True)
    hmean = jnp.concatenate([hm0, hm1], axis=0) * (1.0 / NT)
    s2 = s_ref[...].reshape(B, E * H_EXP)
    ymoe = _dot(s2, w2_ref[...]) * (1.0 / NT)
    gv = g_ref[...].reshape(E, B, NT)
    gs = jnp.transpose(jnp.sum(gv, axis=2))
    ymoe = ymoe + _dot(gs, b2_ref[...]) * (1.0 / NT)
    y = hmean + ymoe
    out_ref[...] = _dot(y, wout_ref[...]) + bout_ref[...]


def kernel(x, U, R, S, W_in, b_in, W_tok, b_tok, W_q, b_q, W_k, b_k, W_v, b_v,
           W_ih, b_ih, W_hh, b_hh, W_g1, b_g1, W_g2, b_g2, W_e1, b_e1,
           W_e2, b_e2, W_out, b_out):
    f32 = jnp.float32
    pe = jnp.asarray(_pos_encoding_np(T, D_MODEL))
    x2 = x.reshape(N, D_IN)
    u3 = U.reshape(SEQ, T, 1)
    rt = jnp.transpose(R.reshape(SEQ, M, T), (0, 2, 1))
    st = jnp.transpose(S.reshape(SEQ, M, T), (0, 2, 1))
    wk_t = W_k.T
    wih_t = W_ih.T
    whh_t = W_hh.T

    def row2(v):
        return v.reshape(1, -1)

    h, proc, gi = pl.pallas_call(
        _pre_kernel,
        grid=(SEQ,),
        in_specs=[
            pl.BlockSpec((T, D_IN), lambda i: (i, 0)),
            pl.BlockSpec((T, D_MODEL), lambda i: (0, 0)),
            pl.BlockSpec((1, T, 1), lambda i: (i, 0, 0)),
            pl.BlockSpec((1, T, M), lambda i: (i, 0, 0)),
            pl.BlockSpec((1, T, M), lambda i: (i, 0, 0)),
            pl.BlockSpec((D_IN, D_MODEL), lambda i: (0, 0)),
            pl.BlockSpec((1, D_MODEL), lambda i: (0, 0)),
            pl.BlockSpec((D_MODEL, TP), lambda i: (0, 0)),
            pl.BlockSpec((1, TP), lambda i: (0, 0)),
            pl.BlockSpec((TP, AK), lambda i: (0, 0)),
            pl.BlockSpec((1, AK), lambda i: (0, 0)),
            pl.BlockSpec((AK, 2), lambda i: (0, 0)),
            pl.BlockSpec((2, AV), lambda i: (0, 0)),
            pl.BlockSpec((1, AV), lambda i: (0, 0)),
            pl.BlockSpec((1 + AV, 3 * GRU_H), lambda i: (0, 0)),
            pl.BlockSpec((1, 3 * GRU_H), lambda i: (0, 0)),
        ],
        out_specs=[
            pl.BlockSpec((T, D_MODEL), lambda i: (i, 0)),
            pl.BlockSpec((T, TP), lambda i: (i, 0)),
            pl.BlockSpec((1, T, 3 * GRU_H), lambda i: (i, 0, 0)),
        ],
        out_shape=[
            jax.ShapeDtypeStruct((N, D_MODEL), f32),
            jax.ShapeDtypeStruct((N, TP), f32),
            jax.ShapeDtypeStruct((SEQ, T, 3 * GRU_H), f32),
        ],
    )(x2, pe, u3, rt, st, W_in, row2(b_in), W_tok, row2(b_tok),
      W_q, row2(b_q), wk_t, W_v, row2(b_v), wih_t, row2(b_ih))

    gi_t = jnp.transpose(gi, (1, 0, 2))
    hs = pl.pallas_call(
        _gru_kernel,
        out_shape=jax.ShapeDtypeStruct((T, SEQ, GRU_H), f32),
    )(gi_t, whh_t, row2(b_hh))
    gru_out = jnp.transpose(hs, (1, 0, 2)).reshape(N, GRU_H)

    gates = pl.pallas_call(
        _route_kernel,
        grid=(N // T,),
        in_specs=[
            pl.BlockSpec((T, TP), lambda i: (i, 0)),
            pl.BlockSpec((T, GRU_H), lambda i: (i, 0)),
            pl.BlockSpec((TP, (TP + GRU_H) // 2), lambda i: (0, 0)),
            pl.BlockSpec((GRU_H, (TP + GRU_H) // 2), lambda i: (0, 0)),
            pl.BlockSpec((1, (TP + GRU_H) // 2), lambda i: (0, 0)),
            pl.BlockSpec(((TP + GRU_H) // 2, E), lambda i: (0, 0)),
            pl.BlockSpec((1, E), lambda i: (0, 0)),
        ],
        out_specs=pl.BlockSpec((T, E), lambda i: (i, 0)),
        out_shape=jax.ShapeDtypeStruct((N, E), f32),
    )(proc, gru_out, W_g1[:TP], W_g1[TP:], row2(b_g1), W_g2, row2(b_g2))

    gates_t = gates.T.reshape(E, 1, N)
    s_pool = pl.pallas_call(
        _expert_kernel,
        grid=(E, B, NBB),
        in_specs=[
            pl.BlockSpec((T, D_MODEL), lambda e, b, k: (b * NBB + k, 0)),
            pl.BlockSpec((1, D_MODEL, H_EXP), lambda e, b, k: (e, 0, 0)),
            pl.BlockSpec((1, 1, H_EXP), lambda e, b, k: (e, 0, 0)),
            pl.BlockSpec((1, 1, T), lambda e, b, k: (e, 0, b * NBB + k)),
        ],
        out_specs=pl.BlockSpec((1, 1, H_EXP), lambda e, b, k: (b * E + e, 0, 0)),
        out_shape=jax.ShapeDtypeStruct((B * E, 1, H_EXP), f32),
    )(h, W_e1, b_e1.reshape(E, 1, H_EXP), gates_t)

    out = pl.pallas_call(
        _final_kernel,
        out_shape=jax.ShapeDtypeStruct((B, N_CLS), f32),
    )(h, s_pool, gates_t, W_e2.reshape(E * H_EXP, D_MODEL), b_e2,
      W_out, row2(b_out))
    return out


# speedup vs baseline: 1.7286x; 1.0035x over previous
import math

import jax
import jax.numpy as jnp
import numpy as np
from jax.experimental import pallas as pl

B, M, T = 2, 4, 512
D_IN = 512
D_MODEL = 512
E, TOPK = 8, 2
H_EXP = 1024
TP = 128
GRU_H = 128
AK = 64
AV = 64
N_CLS = 10
N = B * M * T
SEQ = B * M
NT = M * T
NBB = NT // T
NEG = -1e30


def _pos_encoding_np(t, d):
    position = np.arange(t)[:, None].astype(np.float32)
    div = np.exp(np.arange(0, d, 2).astype(np.float32) * (-math.log(10000.0) / d))
    pe = np.zeros((t, d), dtype=np.float32)
    pe[:, 0::2] = np.sin(position * div)
    pe[:, 1::2] = np.cos(position * div)
    return pe


def _dot(a, b):
    return jnp.dot(a, b, preferred_element_type=jnp.float32)


def _pre_kernel(x_ref, pe_ref, u_ref, r_ref, s_ref,
                w_in_ref, b_in_ref, w_tok_ref, b_tok_ref, w_q_ref, b_q_ref,
                wk_t_ref, w_v_ref, b_v_ref, wih_t_ref, b_ih_ref,
                h16_ref, hsum_ref, proc_ref, gi_ref):
    i = pl.program_id(0)
    m = i % M
    h = _dot(x_ref[...], w_in_ref[...]) + b_in_ref[...] + pe_ref[...]
    h16_ref[...] = h.astype(jnp.bfloat16)
    hsum_ref[0] = jnp.sum(h, axis=0, keepdims=True)
    proc = jnp.maximum(_dot(h, w_tok_ref[...]) + b_tok_ref[...], 0.0)
    proc_ref[...] = proc
    q = _dot(proc, w_q_ref[...]) + b_q_ref[...]
    qk = _dot(q, wk_t_ref[...]) * (1.0 / math.sqrt(AK))
    a_r = qk[:, 0:1]
    a_s = qk[:, 1:2]
    rb = r_ref[0]
    sb = s_ref[0]
    scores = rb * a_r + sb * a_s
    col = jax.lax.broadcasted_iota(jnp.int32, (T, M), 1)
    scores = jnp.where(col == m, NEG, scores)
    w = jnp.exp(scores - jnp.max(scores, axis=1, keepdims=True))
    w = w / jnp.sum(w, axis=1, keepdims=True)
    wr = jnp.sum(w * rb, axis=1, keepdims=True)
    ws = jnp.sum(w * sb, axis=1, keepdims=True)
    c_u = wih_t_ref[0:1, :]
    c_rs = _dot(w_v_ref[...], wih_t_ref[1:, :])
    c_0 = _dot(b_v_ref[...], wih_t_ref[1:, :]) + b_ih_ref[...]
    gi = (u_ref[0] * c_u + wr * c_rs[0:1, :] + ws * c_rs[1:2, :] + c_0)
    gi_ref[0] = gi


def _gru_kernel(gi_ref, whh_t_ref, b_hh_ref, out_ref):
    def step(t, h):
        gi = gi_ref[t]
        gh = _dot(h, whh_t_ref[...]) + b_hh_ref[...]
        r = jax.nn.sigmoid(gi[:, :GRU_H] + gh[:, :GRU_H])
        z = jax.nn.sigmoid(gi[:, GRU_H:2 * GRU_H] + gh[:, GRU_H:2 * GRU_H])
        n = jnp.tanh(gi[:, 2 * GRU_H:] + r * gh[:, 2 * GRU_H:])
        h_new = (1.0 - z) * n + z * h
        out_ref[t] = h_new
        return h_new
    jax.lax.fori_loop(0, T, step, jnp.zeros((SEQ, GRU_H), jnp.float32))


def _route_kernel(proc_ref, gru_ref, wg1a_ref, wg1b_ref, b_g1_ref,
                  w_g2_ref, b_g2_ref, gates_ref):
    hid = jnp.maximum(_dot(proc_ref[...], wg1a_ref[...]) +
                      _dot(gru_ref[...], wg1b_ref[...]) + b_g1_ref[...], 0.0)
    logits = _dot(hid, w_g2_ref[...]) + b_g2_ref[...]
    idx = jax.lax.broadcasted_iota(jnp.int32, (T, E), 1)
    v1 = jnp.max(logits, axis=1, keepdims=True)
    i1 = jnp.min(jnp.where(logits == v1, idx, E), axis=1, keepdims=True)
    masked = jnp.where(idx == i1, NEG, logits)
    v2 = jnp.max(masked, axis=1, keepdims=True)
    i2 = jnp.min(jnp.where(masked == v2, idx, E), axis=1, keepdims=True)
    e2 = jnp.exp(v2 - v1)
    g1 = 1.0 / (1.0 + e2)
    gates_ref[...] = jnp.where(idx == i1, g1, 0.0) + jnp.where(idx == i2, e2 * g1, 0.0)


def _expert_kernel(h16_ref, w1_ref, b1_ref, g_ref, s_ref):
    k = pl.program_id(2)
    eh = jnp.maximum(_dot(h16_ref[...], w1_ref[0]) + b1_ref[0], 0.0)
    contrib = _dot(g_ref[0], eh)

    @pl.when(k == 0)
    def _():
        s_ref[0] = contrib

    @pl.when(k != 0)
    def _():
        s_ref[0] += contrib


def _final_kernel(hsum_ref, s_ref, g_ref, w2_ref, b2_ref, wout_ref, bout_ref, out_ref):
    hs = hsum_ref[...].reshape(B, M, D_MODEL)
    hmean = jnp.sum(hs, axis=1) * (1.0 / NT)
    s2 = s_ref[...].reshape(B, E * H_EXP)
    ymoe = _dot(s2, w2_ref[...]) * (1.0 / NT)
    gv = g_ref[...].reshape(E, B, NT)
    gs = jnp.transpose(jnp.sum(gv, axis=2))
    ymoe = ymoe + _dot(gs, b2_ref[...]) * (1.0 / NT)
    y = hmean + ymoe
    out_ref[...] = _dot(y, wout_ref[...]) + bout_ref[...]


def kernel(x, U, R, S, W_in, b_in, W_tok, b_tok, W_q, b_q, W_k, b_k, W_v, b_v,
           W_ih, b_ih, W_hh, b_hh, W_g1, b_g1, W_g2, b_g2, W_e1, b_e1,
           W_e2, b_e2, W_out, b_out):
    f32 = jnp.float32
    pe = jnp.asarray(_pos_encoding_np(T, D_MODEL))
    x2 = x.reshape(N, D_IN)
    u3 = U.reshape(SEQ, T, 1)
    rt = jnp.transpose(R.reshape(SEQ, M, T), (0, 2, 1))
    st = jnp.transpose(S.reshape(SEQ, M, T), (0, 2, 1))
    wk_t = W_k.T
    wih_t = W_ih.T
    whh_t = W_hh.T

    def row2(v):
        return v.reshape(1, -1)

    h16, hsum, proc, gi = pl.pallas_call(
        _pre_kernel,
        grid=(SEQ,),
        in_specs=[
            pl.BlockSpec((T, D_IN), lambda i: (i, 0)),
            pl.BlockSpec((T, D_MODEL), lambda i: (0, 0)),
            pl.BlockSpec((1, T, 1), lambda i: (i, 0, 0)),
            pl.BlockSpec((1, T, M), lambda i: (i, 0, 0)),
            pl.BlockSpec((1, T, M), lambda i: (i, 0, 0)),
            pl.BlockSpec((D_IN, D_MODEL), lambda i: (0, 0)),
            pl.BlockSpec((1, D_MODEL), lambda i: (0, 0)),
            pl.BlockSpec((D_MODEL, TP), lambda i: (0, 0)),
            pl.BlockSpec((1, TP), lambda i: (0, 0)),
            pl.BlockSpec((TP, AK), lambda i: (0, 0)),
            pl.BlockSpec((1, AK), lambda i: (0, 0)),
            pl.BlockSpec((AK, 2), lambda i: (0, 0)),
            pl.BlockSpec((2, AV), lambda i: (0, 0)),
            pl.BlockSpec((1, AV), lambda i: (0, 0)),
            pl.BlockSpec((1 + AV, 3 * GRU_H), lambda i: (0, 0)),
            pl.BlockSpec((1, 3 * GRU_H), lambda i: (0, 0)),
        ],
        out_specs=[
            pl.BlockSpec((T, D_MODEL), lambda i: (i, 0)),
            pl.BlockSpec((1, 1, D_MODEL), lambda i: (i, 0, 0)),
            pl.BlockSpec((T, TP), lambda i: (i, 0)),
            pl.BlockSpec((1, T, 3 * GRU_H), lambda i: (i, 0, 0)),
        ],
        out_shape=[
            jax.ShapeDtypeStruct((N, D_MODEL), jnp.bfloat16),
            jax.ShapeDtypeStruct((SEQ, 1, D_MODEL), f32),
            jax.ShapeDtypeStruct((N, TP), f32),
            jax.ShapeDtypeStruct((SEQ, T, 3 * GRU_H), f32),
        ],
    )(x2, pe, u3, rt, st, W_in, row2(b_in), W_tok, row2(b_tok),
      W_q, row2(b_q), wk_t, W_v, row2(b_v), wih_t, row2(b_ih))

    gi_t = jnp.transpose(gi, (1, 0, 2))
    hs = pl.pallas_call(
        _gru_kernel,
        out_shape=jax.ShapeDtypeStruct((T, SEQ, GRU_H), f32),
    )(gi_t, whh_t, row2(b_hh))
    gru_out = jnp.transpose(hs, (1, 0, 2)).reshape(N, GRU_H)

    gates = pl.pallas_call(
        _route_kernel,
        grid=(N // T,),
        in_specs=[
            pl.BlockSpec((T, TP), lambda i: (i, 0)),
            pl.BlockSpec((T, GRU_H), lambda i: (i, 0)),
            pl.BlockSpec((TP, (TP + GRU_H) // 2), lambda i: (0, 0)),
            pl.BlockSpec((GRU_H, (TP + GRU_H) // 2), lambda i: (0, 0)),
            pl.BlockSpec((1, (TP + GRU_H) // 2), lambda i: (0, 0)),
            pl.BlockSpec(((TP + GRU_H) // 2, E), lambda i: (0, 0)),
            pl.BlockSpec((1, E), lambda i: (0, 0)),
        ],
        out_specs=pl.BlockSpec((T, E), lambda i: (i, 0)),
        out_shape=jax.ShapeDtypeStruct((N, E), f32),
    )(proc, gru_out, W_g1[:TP], W_g1[TP:], row2(b_g1), W_g2, row2(b_g2))

    gates_t = gates.T.reshape(E, 1, N)
    s_pool = pl.pallas_call(
        _expert_kernel,
        grid=(E, B, NBB),
        in_specs=[
            pl.BlockSpec((T, D_MODEL), lambda e, b, k: (b * NBB + k, 0)),
            pl.BlockSpec((1, D_MODEL, H_EXP), lambda e, b, k: (e, 0, 0)),
            pl.BlockSpec((1, 1, H_EXP), lambda e, b, k: (e, 0, 0)),
            pl.BlockSpec((1, 1, T), lambda e, b, k: (e, 0, b * NBB + k)),
        ],
        out_specs=pl.BlockSpec((1, 1, H_EXP), lambda e, b, k: (b * E + e, 0, 0)),
        out_shape=jax.ShapeDtypeStruct((B * E, 1, H_EXP), f32),
    )(h16, W_e1.astype(jnp.bfloat16), b_e1.reshape(E, 1, H_EXP), gates_t)

    out = pl.pallas_call(
        _final_kernel,
        out_shape=jax.ShapeDtypeStruct((B, N_CLS), f32),
    )(hsum, s_pool, gates_t, W_e2.reshape(E * H_EXP, D_MODEL), b_e2,
      W_out, row2(b_out))
    return out


# speedup vs baseline: 1.7855x; 1.0329x over previous
import math

import jax
import jax.numpy as jnp
import numpy as np
from jax.experimental import pallas as pl

B, M, T = 2, 4, 512
D_IN = 512
D_MODEL = 512
E, TOPK = 8, 2
H_EXP = 1024
TP = 128
GRU_H = 128
AK = 64
AV = 64
N_CLS = 10
N = B * M * T
SEQ = B * M
NT = M * T
NBB = NT // T
NEG = -1e30


def _pos_encoding_np(t, d):
    position = np.arange(t)[:, None].astype(np.float32)
    div = np.exp(np.arange(0, d, 2).astype(np.float32) * (-math.log(10000.0) / d))
    pe = np.zeros((t, d), dtype=np.float32)
    pe[:, 0::2] = np.sin(position * div)
    pe[:, 1::2] = np.cos(position * div)
    return pe


def _dot(a, b):
    return jnp.dot(a, b, preferred_element_type=jnp.float32)


def _pre_kernel(x_ref, pe_ref, u_ref, r_ref, s_ref,
                w_in_ref, b_in_ref, w_tok_ref, b_tok_ref, w_q_ref, b_q_ref,
                wk_t_ref, w_v_ref, b_v_ref, wih_t_ref, b_ih_ref,
                h16_ref, hsum_ref, proc_ref, gi_ref):
    i = pl.program_id(0)
    m = i % M
    h = _dot(x_ref[...], w_in_ref[...]) + b_in_ref[...] + pe_ref[...]
    h16_ref[...] = h.astype(jnp.bfloat16)
    hsum_ref[0] = jnp.sum(h, axis=0, keepdims=True)
    proc = jnp.maximum(_dot(h, w_tok_ref[...]) + b_tok_ref[...], 0.0)
    proc_ref[...] = proc
    q = _dot(proc, w_q_ref[...]) + b_q_ref[...]
    qk = _dot(q, wk_t_ref[...]) * (1.0 / math.sqrt(AK))
    a_r = qk[:, 0:1]
    a_s = qk[:, 1:2]
    rb = r_ref[0]
    sb = s_ref[0]
    scores = rb * a_r + sb * a_s
    col = jax.lax.broadcasted_iota(jnp.int32, (T, M), 1)
    scores = jnp.where(col == m, NEG, scores)
    w = jnp.exp(scores - jnp.max(scores, axis=1, keepdims=True))
    w = w / jnp.sum(w, axis=1, keepdims=True)
    wr = jnp.sum(w * rb, axis=1, keepdims=True)
    ws = jnp.sum(w * sb, axis=1, keepdims=True)
    c_u = wih_t_ref[0:1, :]
    c_rs = _dot(w_v_ref[...], wih_t_ref[1:, :])
    c_0 = _dot(b_v_ref[...], wih_t_ref[1:, :]) + b_ih_ref[...]
    gi = (u_ref[0] * c_u + wr * c_rs[0:1, :] + ws * c_rs[1:2, :] + c_0)
    gi_ref[0] = gi


def _gru_kernel(gi_ref, whh_t_ref, b_hh_ref, out_ref):
    def step(t, h):
        gi = gi_ref[t]
        gh = _dot(h, whh_t_ref[...]) + b_hh_ref[...]
        r = jax.nn.sigmoid(gi[:, :GRU_H] + gh[:, :GRU_H])
        z = jax.nn.sigmoid(gi[:, GRU_H:2 * GRU_H] + gh[:, GRU_H:2 * GRU_H])
        n = jnp.tanh(gi[:, 2 * GRU_H:] + r * gh[:, 2 * GRU_H:])
        h_new = (1.0 - z) * n + z * h
        out_ref[t] = h_new
        return h_new

    def step4(j, h):
        t = j * 4
        h = step(t, h)
        h = step(t + 1, h)
        h = step(t + 2, h)
        return step(t + 3, h)

    jax.lax.fori_loop(0, T // 4, step4, jnp.zeros((SEQ, GRU_H), jnp.float32))


def _route_kernel(proc_ref, gru_ref, wg1a_ref, wg1b_ref, b_g1_ref,
                  w_g2_ref, b_g2_ref, gates_ref):
    hid = jnp.maximum(_dot(proc_ref[...], wg1a_ref[...]) +
                      _dot(gru_ref[...], wg1b_ref[...]) + b_g1_ref[...], 0.0)
    logits = _dot(hid, w_g2_ref[...]) + b_g2_ref[...]
    idx = jax.lax.broadcasted_iota(jnp.int32, (T, E), 1)
    v1 = jnp.max(logits, axis=1, keepdims=True)
    i1 = jnp.min(jnp.where(logits == v1, idx, E), axis=1, keepdims=True)
    masked = jnp.where(idx == i1, NEG, logits)
    v2 = jnp.max(masked, axis=1, keepdims=True)
    i2 = jnp.min(jnp.where(masked == v2, idx, E), axis=1, keepdims=True)
    e2 = jnp.exp(v2 - v1)
    g1 = 1.0 / (1.0 + e2)
    gates_ref[...] = jnp.where(idx == i1, g1, 0.0) + jnp.where(idx == i2, e2 * g1, 0.0)


def _expert_kernel(h16_ref, w1_ref, b1_ref, g_ref, s_ref):
    k = pl.program_id(2)
    eh = jnp.maximum(_dot(h16_ref[...], w1_ref[0]) + b1_ref[0], 0.0)
    contrib = _dot(g_ref[0], eh)

    @pl.when(k == 0)
    def _():
        s_ref[0] = contrib

    @pl.when(k != 0)
    def _():
        s_ref[0] += contrib


def _final_kernel(hsum_ref, s_ref, g_ref, w2_ref, b2_ref, wout_ref, bout_ref, out_ref):
    hs = hsum_ref[...].reshape(B, M, D_MODEL)
    hmean = jnp.sum(hs, axis=1) * (1.0 / NT)
    s2 = s_ref[...].reshape(B, E * H_EXP)
    ymoe = _dot(s2, w2_ref[...]) * (1.0 / NT)
    gv = g_ref[...].reshape(E, B, NT)
    gs = jnp.transpose(jnp.sum(gv, axis=2))
    ymoe = ymoe + _dot(gs, b2_ref[...]) * (1.0 / NT)
    y = hmean + ymoe
    out_ref[...] = _dot(y, wout_ref[...]) + bout_ref[...]


def kernel(x, U, R, S, W_in, b_in, W_tok, b_tok, W_q, b_q, W_k, b_k, W_v, b_v,
           W_ih, b_ih, W_hh, b_hh, W_g1, b_g1, W_g2, b_g2, W_e1, b_e1,
           W_e2, b_e2, W_out, b_out):
    f32 = jnp.float32
    pe = jnp.asarray(_pos_encoding_np(T, D_MODEL))
    x2 = x.reshape(N, D_IN)
    u3 = U.reshape(SEQ, T, 1)
    rt = jnp.transpose(R.reshape(SEQ, M, T), (0, 2, 1))
    st = jnp.transpose(S.reshape(SEQ, M, T), (0, 2, 1))
    wk_t = W_k.T
    wih_t = W_ih.T
    whh_t = W_hh.T

    def row2(v):
        return v.reshape(1, -1)

    h16, hsum, proc, gi = pl.pallas_call(
        _pre_kernel,
        grid=(SEQ,),
        in_specs=[
            pl.BlockSpec((T, D_IN), lambda i: (i, 0)),
            pl.BlockSpec((T, D_MODEL), lambda i: (0, 0)),
            pl.BlockSpec((1, T, 1), lambda i: (i, 0, 0)),
            pl.BlockSpec((1, T, M), lambda i: (i, 0, 0)),
            pl.BlockSpec((1, T, M), lambda i: (i, 0, 0)),
            pl.BlockSpec((D_IN, D_MODEL), lambda i: (0, 0)),
            pl.BlockSpec((1, D_MODEL), lambda i: (0, 0)),
            pl.BlockSpec((D_MODEL, TP), lambda i: (0, 0)),
            pl.BlockSpec((1, TP), lambda i: (0, 0)),
            pl.BlockSpec((TP, AK), lambda i: (0, 0)),
            pl.BlockSpec((1, AK), lambda i: (0, 0)),
            pl.BlockSpec((AK, 2), lambda i: (0, 0)),
            pl.BlockSpec((2, AV), lambda i: (0, 0)),
            pl.BlockSpec((1, AV), lambda i: (0, 0)),
            pl.BlockSpec((1 + AV, 3 * GRU_H), lambda i: (0, 0)),
            pl.BlockSpec((1, 3 * GRU_H), lambda i: (0, 0)),
        ],
        out_specs=[
            pl.BlockSpec((T, D_MODEL), lambda i: (i, 0)),
            pl.BlockSpec((1, 1, D_MODEL), lambda i: (i, 0, 0)),
            pl.BlockSpec((T, TP), lambda i: (i, 0)),
            pl.BlockSpec((1, T, 3 * GRU_H), lambda i: (i, 0, 0)),
        ],
        out_shape=[
            jax.ShapeDtypeStruct((N, D_MODEL), jnp.bfloat16),
            jax.ShapeDtypeStruct((SEQ, 1, D_MODEL), f32),
            jax.ShapeDtypeStruct((N, TP), f32),
            jax.ShapeDtypeStruct((SEQ, T, 3 * GRU_H), f32),
        ],
    )(x2, pe, u3, rt, st, W_in, row2(b_in), W_tok, row2(b_tok),
      W_q, row2(b_q), wk_t, W_v, row2(b_v), wih_t, row2(b_ih))

    gi_t = jnp.transpose(gi, (1, 0, 2))
    hs = pl.pallas_call(
        _gru_kernel,
        out_shape=jax.ShapeDtypeStruct((T, SEQ, GRU_H), f32),
    )(gi_t, whh_t, row2(b_hh))
    gru_out = jnp.transpose(hs, (1, 0, 2)).reshape(N, GRU_H)

    gates = pl.pallas_call(
        _route_kernel,
        grid=(N // T,),
        in_specs=[
            pl.BlockSpec((T, TP), lambda i: (i, 0)),
            pl.BlockSpec((T, GRU_H), lambda i: (i, 0)),
            pl.BlockSpec((TP, (TP + GRU_H) // 2), lambda i: (0, 0)),
            pl.BlockSpec((GRU_H, (TP + GRU_H) // 2), lambda i: (0, 0)),
            pl.BlockSpec((1, (TP + GRU_H) // 2), lambda i: (0, 0)),
            pl.BlockSpec(((TP + GRU_H) // 2, E), lambda i: (0, 0)),
            pl.BlockSpec((1, E), lambda i: (0, 0)),
        ],
        out_specs=pl.BlockSpec((T, E), lambda i: (i, 0)),
        out_shape=jax.ShapeDtypeStruct((N, E), f32),
    )(proc, gru_out, W_g1[:TP], W_g1[TP:], row2(b_g1), W_g2, row2(b_g2))

    gates_t = gates.T.reshape(E, 1, N)
    s_pool = pl.pallas_call(
        _expert_kernel,
        grid=(E, B, NBB),
        in_specs=[
            pl.BlockSpec((T, D_MODEL), lambda e, b, k: (b * NBB + k, 0)),
            pl.BlockSpec((1, D_MODEL, H_EXP), lambda e, b, k: (e, 0, 0)),
            pl.BlockSpec((1, 1, H_EXP), lambda e, b, k: (e, 0, 0)),
            pl.BlockSpec((1, 1, T), lambda e, b, k: (e, 0, b * NBB + k)),
        ],
        out_specs=pl.BlockSpec((1, 1, H_EXP), lambda e, b, k: (b * E + e, 0, 0)),
        out_shape=jax.ShapeDtypeStruct((B * E, 1, H_EXP), f32),
    )(h16, W_e1.astype(jnp.bfloat16), b_e1.reshape(E, 1, H_EXP), gates_t)

    out = pl.pallas_call(
        _final_kernel,
        out_shape=jax.ShapeDtypeStruct((B, N_CLS), f32),
    )(hsum, s_pool, gates_t, W_e2.reshape(E * H_EXP, D_MODEL), b_e2,
      W_out, row2(b_out))
    return out


# speedup vs baseline: 1.8053x; 1.0111x over previous
import math

import jax
import jax.numpy as jnp
import numpy as np
from jax.experimental import pallas as pl

B, M, T = 2, 4, 512
D_IN = 512
D_MODEL = 512
E, TOPK = 8, 2
H_EXP = 1024
TP = 128
GRU_H = 128
AK = 64
AV = 64
N_CLS = 10
N = B * M * T
SEQ = B * M
NT = M * T
NBB = NT // T
NEG = -1e30


def _pos_encoding_np(t, d):
    position = np.arange(t)[:, None].astype(np.float32)
    div = np.exp(np.arange(0, d, 2).astype(np.float32) * (-math.log(10000.0) / d))
    pe = np.zeros((t, d), dtype=np.float32)
    pe[:, 0::2] = np.sin(position * div)
    pe[:, 1::2] = np.cos(position * div)
    return pe


def _dot(a, b):
    return jnp.dot(a, b, preferred_element_type=jnp.float32)


def _pre_kernel(x_ref, pe_ref, u_ref, r_ref, s_ref,
                w_in_ref, b_in_ref, w_tok_ref, b_tok_ref, w_q_ref, b_q_ref,
                wk_t_ref, w_v_ref, b_v_ref, wih_t_ref, b_ih_ref,
                h16_ref, hsum_ref, proc_ref, gi_ref):
    i = pl.program_id(0)
    m = i % M
    h = _dot(x_ref[...], w_in_ref[...]) + b_in_ref[...] + pe_ref[...]
    h16_ref[...] = h.astype(jnp.bfloat16)
    hsum_ref[0] = jnp.sum(h, axis=0, keepdims=True)
    proc = jnp.maximum(_dot(h, w_tok_ref[...]) + b_tok_ref[...], 0.0)
    proc_ref[...] = proc
    q = _dot(proc, w_q_ref[...]) + b_q_ref[...]
    qk = _dot(q, wk_t_ref[...]) * (1.0 / math.sqrt(AK))
    a_r = qk[:, 0:1]
    a_s = qk[:, 1:2]
    rb = r_ref[0]
    sb = s_ref[0]
    scores = rb * a_r + sb * a_s
    col = jax.lax.broadcasted_iota(jnp.int32, (T, M), 1)
    scores = jnp.where(col == m, NEG, scores)
    w = jnp.exp(scores - jnp.max(scores, axis=1, keepdims=True))
    w = w / jnp.sum(w, axis=1, keepdims=True)
    wr = jnp.sum(w * rb, axis=1, keepdims=True)
    ws = jnp.sum(w * sb, axis=1, keepdims=True)
    c_u = wih_t_ref[0:1, :]
    c_rs = _dot(w_v_ref[...], wih_t_ref[1:, :])
    c_0 = _dot(b_v_ref[...], wih_t_ref[1:, :]) + b_ih_ref[...]
    gi = (u_ref[0] * c_u + wr * c_rs[0:1, :] + ws * c_rs[1:2, :] + c_0)
    gi_ref[0] = gi


def _gru_kernel(gi_ref, whh_t_ref, b_hhn_ref, out_ref):
    def step(t, h):
        gi = gi_ref[t]
        gh = _dot(h, whh_t_ref[...])
        rz = jnp.tanh((gi[:, :2 * GRU_H] + gh[:, :2 * GRU_H]) * 0.5) * 0.5 + 0.5
        r = rz[:, :GRU_H]
        z = rz[:, GRU_H:]
        n = jnp.tanh(gi[:, 2 * GRU_H:] + r * (gh[:, 2 * GRU_H:] + b_hhn_ref[...]))
        h_new = n + z * (h - n)
        out_ref[t] = h_new
        return h_new

    def step4(j, h):
        t = j * 4
        h = step(t, h)
        h = step(t + 1, h)
        h = step(t + 2, h)
        return step(t + 3, h)

    jax.lax.fori_loop(0, T // 4, step4, jnp.zeros((SEQ, GRU_H), jnp.float32))


def _route_kernel(proc_ref, gru_ref, wg1a_ref, wg1b_ref, b_g1_ref,
                  w_g2_ref, b_g2_ref, gates_ref):
    hid = jnp.maximum(_dot(proc_ref[...], wg1a_ref[...]) +
                      _dot(gru_ref[...], wg1b_ref[...]) + b_g1_ref[...], 0.0)
    logits = _dot(hid, w_g2_ref[...]) + b_g2_ref[...]
    idx = jax.lax.broadcasted_iota(jnp.int32, (T, E), 1)
    v1 = jnp.max(logits, axis=1, keepdims=True)
    i1 = jnp.min(jnp.where(logits == v1, idx, E), axis=1, keepdims=True)
    masked = jnp.where(idx == i1, NEG, logits)
    v2 = jnp.max(masked, axis=1, keepdims=True)
    i2 = jnp.min(jnp.where(masked == v2, idx, E), axis=1, keepdims=True)
    e2 = jnp.exp(v2 - v1)
    g1 = 1.0 / (1.0 + e2)
    gates_ref[...] = jnp.where(idx == i1, g1, 0.0) + jnp.where(idx == i2, e2 * g1, 0.0)


def _expert_kernel(h16_ref, w1_ref, b1_ref, g_ref, s_ref):
    k = pl.program_id(2)
    eh = jnp.maximum(_dot(h16_ref[...], w1_ref[0]) + b1_ref[0], 0.0)
    contrib = _dot(g_ref[0], eh)

    @pl.when(k == 0)
    def _():
        s_ref[0] = contrib

    @pl.when(k != 0)
    def _():
        s_ref[0] += contrib


def _final_kernel(hsum_ref, s_ref, g_ref, w2_ref, b2_ref, wout_ref, bout_ref, out_ref):
    hs = hsum_ref[...].reshape(B, M, D_MODEL)
    hmean = jnp.sum(hs, axis=1) * (1.0 / NT)
    s2 = s_ref[...].reshape(B, E * H_EXP)
    ymoe = _dot(s2, w2_ref[...]) * (1.0 / NT)
    gv = g_ref[...].reshape(E, B, NT)
    gs = jnp.transpose(jnp.sum(gv, axis=2))
    ymoe = ymoe + _dot(gs, b2_ref[...]) * (1.0 / NT)
    y = hmean + ymoe
    out_ref[...] = _dot(y, wout_ref[...]) + bout_ref[...]


def kernel(x, U, R, S, W_in, b_in, W_tok, b_tok, W_q, b_q, W_k, b_k, W_v, b_v,
           W_ih, b_ih, W_hh, b_hh, W_g1, b_g1, W_g2, b_g2, W_e1, b_e1,
           W_e2, b_e2, W_out, b_out):
    f32 = jnp.float32
    pe = jnp.asarray(_pos_encoding_np(T, D_MODEL))
    x2 = x.reshape(N, D_IN)
    u3 = U.reshape(SEQ, T, 1)
    rt = jnp.transpose(R.reshape(SEQ, M, T), (0, 2, 1))
    st = jnp.transpose(S.reshape(SEQ, M, T), (0, 2, 1))
    wk_t = W_k.T
    wih_t = W_ih.T
    whh_t = W_hh.T

    def row2(v):
        return v.reshape(1, -1)

    h16, hsum, proc, gi = pl.pallas_call(
        _pre_kernel,
        grid=(SEQ,),
        in_specs=[
            pl.BlockSpec((T, D_IN), lambda i: (i, 0)),
            pl.BlockSpec((T, D_MODEL), lambda i: (0, 0)),
            pl.BlockSpec((1, T, 1), lambda i: (i, 0, 0)),
            pl.BlockSpec((1, T, M), lambda i: (i, 0, 0)),
            pl.BlockSpec((1, T, M), lambda i: (i, 0, 0)),
            pl.BlockSpec((D_IN, D_MODEL), lambda i: (0, 0)),
            pl.BlockSpec((1, D_MODEL), lambda i: (0, 0)),
            pl.BlockSpec((D_MODEL, TP), lambda i: (0, 0)),
            pl.BlockSpec((1, TP), lambda i: (0, 0)),
            pl.BlockSpec((TP, AK), lambda i: (0, 0)),
            pl.BlockSpec((1, AK), lambda i: (0, 0)),
            pl.BlockSpec((AK, 2), lambda i: (0, 0)),
            pl.BlockSpec((2, AV), lambda i: (0, 0)),
            pl.BlockSpec((1, AV), lambda i: (0, 0)),
            pl.BlockSpec((1 + AV, 3 * GRU_H), lambda i: (0, 0)),
            pl.BlockSpec((1, 3 * GRU_H), lambda i: (0, 0)),
        ],
        out_specs=[
            pl.BlockSpec((T, D_MODEL), lambda i: (i, 0)),
            pl.BlockSpec((1, 1, D_MODEL), lambda i: (i, 0, 0)),
            pl.BlockSpec((T, TP), lambda i: (i, 0)),
            pl.BlockSpec((1, T, 3 * GRU_H), lambda i: (i, 0, 0)),
        ],
        out_shape=[
            jax.ShapeDtypeStruct((N, D_MODEL), jnp.bfloat16),
            jax.ShapeDtypeStruct((SEQ, 1, D_MODEL), f32),
            jax.ShapeDtypeStruct((N, TP), f32),
            jax.ShapeDtypeStruct((SEQ, T, 3 * GRU_H), f32),
        ],
    )(x2, pe, u3, rt, st, W_in, row2(b_in), W_tok, row2(b_tok),
      W_q, row2(b_q), wk_t, W_v, row2(b_v), wih_t,
      row2(b_ih + jnp.concatenate([b_hh[:2 * GRU_H],
                                   jnp.zeros((GRU_H,), jnp.float32)])))

    gi_t = jnp.transpose(gi, (1, 0, 2))
    hs = pl.pallas_call(
        _gru_kernel,
        out_shape=jax.ShapeDtypeStruct((T, SEQ, GRU_H), f32),
    )(gi_t, whh_t, row2(b_hh[2 * GRU_H:]))
    gru_out = jnp.transpose(hs, (1, 0, 2)).reshape(N, GRU_H)

    gates = pl.pallas_call(
        _route_kernel,
        grid=(N // T,),
        in_specs=[
            pl.BlockSpec((T, TP), lambda i: (i, 0)),
            pl.BlockSpec((T, GRU_H), lambda i: (i, 0)),
            pl.BlockSpec((TP, (TP + GRU_H) // 2), lambda i: (0, 0)),
            pl.BlockSpec((GRU_H, (TP + GRU_H) // 2), lambda i: (0, 0)),
            pl.BlockSpec((1, (TP + GRU_H) // 2), lambda i: (0, 0)),
            pl.BlockSpec(((TP + GRU_H) // 2, E), lambda i: (0, 0)),
            pl.BlockSpec((1, E), lambda i: (0, 0)),
        ],
        out_specs=pl.BlockSpec((T, E), lambda i: (i, 0)),
        out_shape=jax.ShapeDtypeStruct((N, E), f32),
    )(proc, gru_out, W_g1[:TP], W_g1[TP:], row2(b_g1), W_g2, row2(b_g2))

    gates_t = gates.T.reshape(E, 1, N)
    s_pool = pl.pallas_call(
        _expert_kernel,
        grid=(E, B, NBB),
        in_specs=[
            pl.BlockSpec((T, D_MODEL), lambda e, b, k: (b * NBB + k, 0)),
            pl.BlockSpec((1, D_MODEL, H_EXP), lambda e, b, k: (e, 0, 0)),
            pl.BlockSpec((1, 1, H_EXP), lambda e, b, k: (e, 0, 0)),
            pl.BlockSpec((1, 1, T), lambda e, b, k: (e, 0, b * NBB + k)),
        ],
        out_specs=pl.BlockSpec((1, 1, H_EXP), lambda e, b, k: (b * E + e, 0, 0)),
        out_shape=jax.ShapeDtypeStruct((B * E, 1, H_EXP), f32),
    )(h16, W_e1.astype(jnp.bfloat16), b_e1.reshape(E, 1, H_EXP), gates_t)

    out = pl.pallas_call(
        _final_kernel,
        out_shape=jax.ShapeDtypeStruct((B, N_CLS), f32),
    )(hsum, s_pool, gates_t, W_e2.reshape(E * H_EXP, D_MODEL), b_e2,
      W_out, row2(b_out))
    return out


# speedup vs baseline: 1.8639x; 1.0324x over previous
import math

import jax
import jax.numpy as jnp
import numpy as np
from jax.experimental import pallas as pl

B, M, T = 2, 4, 512
D_IN = 512
D_MODEL = 512
E, TOPK = 8, 2
H_EXP = 1024
TP = 128
GRU_H = 128
AK = 64
AV = 64
N_CLS = 10
N = B * M * T
SEQ = B * M
NT = M * T
NBB = NT // T
NEG = -1e30


def _pos_encoding_np(t, d):
    position = np.arange(t)[:, None].astype(np.float32)
    div = np.exp(np.arange(0, d, 2).astype(np.float32) * (-math.log(10000.0) / d))
    pe = np.zeros((t, d), dtype=np.float32)
    pe[:, 0::2] = np.sin(position * div)
    pe[:, 1::2] = np.cos(position * div)
    return pe


def _dot(a, b):
    return jnp.dot(a, b, preferred_element_type=jnp.float32)


def _pre_kernel(x_ref, pe_ref, u_ref, r_ref, s_ref,
                w_in_ref, b_in_ref, w_tok_ref, b_tok_ref, w_q_ref, b_q_ref,
                wk_t_ref, w_v_ref, b_v_ref, wih_t_ref, b_ih_ref,
                h16_ref, hsum_ref, proc_ref, gi_ref):
    i = pl.program_id(0)
    m = i % M
    h = _dot(x_ref[...], w_in_ref[...]) + b_in_ref[...] + pe_ref[...]
    h16_ref[...] = h.astype(jnp.bfloat16)
    hsum_ref[0] = jnp.sum(h, axis=0, keepdims=True)
    proc = jnp.maximum(_dot(h, w_tok_ref[...]) + b_tok_ref[...], 0.0)
    proc_ref[...] = proc
    q = _dot(proc, w_q_ref[...]) + b_q_ref[...]
    qk = _dot(q, wk_t_ref[...]) * (1.0 / math.sqrt(AK))
    a_r = qk[:, 0:1]
    a_s = qk[:, 1:2]
    rb = r_ref[0]
    sb = s_ref[0]
    scores = rb * a_r + sb * a_s
    col = jax.lax.broadcasted_iota(jnp.int32, (T, M), 1)
    scores = jnp.where(col == m, NEG, scores)
    w = jnp.exp(scores - jnp.max(scores, axis=1, keepdims=True))
    w = w / jnp.sum(w, axis=1, keepdims=True)
    wr = jnp.sum(w * rb, axis=1, keepdims=True)
    ws = jnp.sum(w * sb, axis=1, keepdims=True)
    c_u = wih_t_ref[0:1, :]
    c_rs = _dot(w_v_ref[...], wih_t_ref[1:, :])
    c_0 = _dot(b_v_ref[...], wih_t_ref[1:, :]) + b_ih_ref[...]
    gi = (u_ref[0] * c_u + wr * c_rs[0:1, :] + ws * c_rs[1:2, :] + c_0)
    gi_ref[0] = gi


def _gru_kernel(gi_ref, whh_t_ref, b_hhn_ref, out_ref):
    def step(t, h):
        gi = gi_ref[:, t, :]
        gh = _dot(h, whh_t_ref[...])
        rz = jnp.tanh((gi[:, :2 * GRU_H] + gh[:, :2 * GRU_H]) * 0.5) * 0.5 + 0.5
        r = rz[:, :GRU_H]
        z = rz[:, GRU_H:]
        n = jnp.tanh(gi[:, 2 * GRU_H:] + r * (gh[:, 2 * GRU_H:] + b_hhn_ref[...]))
        h_new = n + z * (h - n)
        out_ref[:, t, :] = h_new
        return h_new

    def step4(j, h):
        t = j * 4
        h = step(t, h)
        h = step(t + 1, h)
        h = step(t + 2, h)
        return step(t + 3, h)

    jax.lax.fori_loop(0, T // 4, step4, jnp.zeros((SEQ, GRU_H), jnp.float32))


def _route_kernel(proc_ref, gru_ref, wg1a_ref, wg1b_ref, b_g1_ref,
                  w_g2_ref, b_g2_ref, gates_ref):
    hid = jnp.maximum(_dot(proc_ref[...], wg1a_ref[...]) +
                      _dot(gru_ref[...], wg1b_ref[...]) + b_g1_ref[...], 0.0)
    logits = _dot(hid, w_g2_ref[...]) + b_g2_ref[...]
    idx = jax.lax.broadcasted_iota(jnp.int32, (T, E), 1)
    v1 = jnp.max(logits, axis=1, keepdims=True)
    i1 = jnp.min(jnp.where(logits == v1, idx, E), axis=1, keepdims=True)
    masked = jnp.where(idx == i1, NEG, logits)
    v2 = jnp.max(masked, axis=1, keepdims=True)
    i2 = jnp.min(jnp.where(masked == v2, idx, E), axis=1, keepdims=True)
    e2 = jnp.exp(v2 - v1)
    g1 = 1.0 / (1.0 + e2)
    gates_ref[...] = jnp.where(idx == i1, g1, 0.0) + jnp.where(idx == i2, e2 * g1, 0.0)


def _expert_kernel(h16_ref, w1_ref, b1_ref, g_ref, s_ref):
    k = pl.program_id(2)
    eh = jnp.maximum(_dot(h16_ref[...], w1_ref[0]) + b1_ref[0], 0.0)
    contrib = _dot(g_ref[0], eh)

    @pl.when(k == 0)
    def _():
        s_ref[0] = contrib

    @pl.when(k != 0)
    def _():
        s_ref[0] += contrib


def _final_kernel(hsum_ref, s_ref, g_ref, w2_ref, b2_ref, wout_ref, bout_ref, out_ref):
    hs = hsum_ref[...].reshape(B, M, D_MODEL)
    hmean = jnp.sum(hs, axis=1) * (1.0 / NT)
    s2 = s_ref[...].reshape(B, E * H_EXP)
    ymoe = _dot(s2, w2_ref[...]) * (1.0 / NT)
    gv = g_ref[...].reshape(E, B, NT)
    gs = jnp.transpose(jnp.sum(gv, axis=2))
    ymoe = ymoe + _dot(gs, b2_ref[...]) * (1.0 / NT)
    y = hmean + ymoe
    out_ref[...] = _dot(y, wout_ref[...]) + bout_ref[...]


def kernel(x, U, R, S, W_in, b_in, W_tok, b_tok, W_q, b_q, W_k, b_k, W_v, b_v,
           W_ih, b_ih, W_hh, b_hh, W_g1, b_g1, W_g2, b_g2, W_e1, b_e1,
           W_e2, b_e2, W_out, b_out):
    f32 = jnp.float32
    pe = jnp.asarray(_pos_encoding_np(T, D_MODEL))
    x2 = x.reshape(N, D_IN)
    u3 = U.reshape(SEQ, T, 1)
    rt = jnp.transpose(R.reshape(SEQ, M, T), (0, 2, 1))
    st = jnp.transpose(S.reshape(SEQ, M, T), (0, 2, 1))
    wk_t = W_k.T
    wih_t = W_ih.T
    whh_t = W_hh.T

    def row2(v):
        return v.reshape(1, -1)

    h16, hsum, proc, gi = pl.pallas_call(
        _pre_kernel,
        grid=(SEQ,),
        in_specs=[
            pl.BlockSpec((T, D_IN), lambda i: (i, 0)),
            pl.BlockSpec((T, D_MODEL), lambda i: (0, 0)),
            pl.BlockSpec((1, T, 1), lambda i: (i, 0, 0)),
            pl.BlockSpec((1, T, M), lambda i: (i, 0, 0)),
            pl.BlockSpec((1, T, M), lambda i: (i, 0, 0)),
            pl.BlockSpec((D_IN, D_MODEL), lambda i: (0, 0)),
            pl.BlockSpec((1, D_MODEL), lambda i: (0, 0)),
            pl.BlockSpec((D_MODEL, TP), lambda i: (0, 0)),
            pl.BlockSpec((1, TP), lambda i: (0, 0)),
            pl.BlockSpec((TP, AK), lambda i: (0, 0)),
            pl.BlockSpec((1, AK), lambda i: (0, 0)),
            pl.BlockSpec((AK, 2), lambda i: (0, 0)),
            pl.BlockSpec((2, AV), lambda i: (0, 0)),
            pl.BlockSpec((1, AV), lambda i: (0, 0)),
            pl.BlockSpec((1 + AV, 3 * GRU_H), lambda i: (0, 0)),
            pl.BlockSpec((1, 3 * GRU_H), lambda i: (0, 0)),
        ],
        out_specs=[
            pl.BlockSpec((T, D_MODEL), lambda i: (i, 0)),
            pl.BlockSpec((1, 1, D_MODEL), lambda i: (i, 0, 0)),
            pl.BlockSpec((T, TP), lambda i: (i, 0)),
            pl.BlockSpec((1, T, 3 * GRU_H), lambda i: (i, 0, 0)),
        ],
        out_shape=[
            jax.ShapeDtypeStruct((N, D_MODEL), jnp.bfloat16),
            jax.ShapeDtypeStruct((SEQ, 1, D_MODEL), f32),
            jax.ShapeDtypeStruct((N, TP), f32),
            jax.ShapeDtypeStruct((SEQ, T, 3 * GRU_H), f32),
        ],
    )(x2, pe, u3, rt, st, W_in, row2(b_in), W_tok, row2(b_tok),
      W_q, row2(b_q), wk_t, W_v, row2(b_v), wih_t,
      row2(b_ih + jnp.concatenate([b_hh[:2 * GRU_H],
                                   jnp.zeros((GRU_H,), jnp.float32)])))

    hs = pl.pallas_call(
        _gru_kernel,
        out_shape=jax.ShapeDtypeStruct((SEQ, T, GRU_H), f32),
    )(gi, whh_t, row2(b_hh[2 * GRU_H:]))
    gru_out = hs.reshape(N, GRU_H)

    gates = pl.pallas_call(
        _route_kernel,
        grid=(N // T,),
        in_specs=[
            pl.BlockSpec((T, TP), lambda i: (i, 0)),
            pl.BlockSpec((T, GRU_H), lambda i: (i, 0)),
            pl.BlockSpec((TP, (TP + GRU_H) // 2), lambda i: (0, 0)),
            pl.BlockSpec((GRU_H, (TP + GRU_H) // 2), lambda i: (0, 0)),
            pl.BlockSpec((1, (TP + GRU_H) // 2), lambda i: (0, 0)),
            pl.BlockSpec(((TP + GRU_H) // 2, E), lambda i: (0, 0)),
            pl.BlockSpec((1, E), lambda i: (0, 0)),
        ],
        out_specs=pl.BlockSpec((T, E), lambda i: (i, 0)),
        out_shape=jax.ShapeDtypeStruct((N, E), f32),
    )(proc, gru_out, W_g1[:TP], W_g1[TP:], row2(b_g1), W_g2, row2(b_g2))

    gates_t = gates.T.reshape(E, 1, N)
    s_pool = pl.pallas_call(
        _expert_kernel,
        grid=(E, B, NBB),
        in_specs=[
            pl.BlockSpec((T, D_MODEL), lambda e, b, k: (b * NBB + k, 0)),
            pl.BlockSpec((1, D_MODEL, H_EXP), lambda e, b, k: (e, 0, 0)),
            pl.BlockSpec((1, 1, H_EXP), lambda e, b, k: (e, 0, 0)),
            pl.BlockSpec((1, 1, T), lambda e, b, k: (e, 0, b * NBB + k)),
        ],
        out_specs=pl.BlockSpec((1, 1, H_EXP), lambda e, b, k: (b * E + e, 0, 0)),
        out_shape=jax.ShapeDtypeStruct((B * E, 1, H_EXP), f32),
    )(h16, W_e1.astype(jnp.bfloat16), b_e1.reshape(E, 1, H_EXP), gates_t)

    out = pl.pallas_call(
        _final_kernel,
        out_shape=jax.ShapeDtypeStruct((B, N_CLS), f32),
    )(hsum, s_pool, gates_t, W_e2.reshape(E * H_EXP, D_MODEL), b_e2,
      W_out, row2(b_out))
    return out
